# baseline (speedup 1.0000x reference)
import functools
import math

import jax
import jax.numpy as jnp
from jax import lax
from jax.experimental import pallas as pl
from jax.experimental.pallas import tpu as pltpu
from jax.experimental.pallas import tpu_sc as plsc

EMB = 64
SCALE = math.sqrt(EMB)

N0 = 16384
N1 = 50
B = N0 * N1
NT = 1000000
NC, NS, L = 2, 16, 16
NW = NC * NS
PER_W = B // NW
C = 256
NPAIR = PER_W // C // 2

CB = 2048


def _transpose_body(tT_ref, out_ref):
    blk = tT_ref[...]
    e = lax.broadcasted_iota(jnp.int32, (EMB, EMB), 0)
    j = lax.broadcasted_iota(jnp.int32, (EMB, EMB), 1)
    ident = jnp.where(e == j, SCALE, 0.0).astype(jnp.float32)
    res = lax.dot_general(blk, ident, (((0,), (0,)), ((), ())),
                          precision=lax.Precision.HIGHEST,
                          preferred_element_type=jnp.float32)
    out_ref[:, 0:EMB] = res


def _body(tok_hbm, table_hbm, out_hbm, idx_v, r0, r1, g0, g1):
    wid = lax.axis_index("s") * NC + lax.axis_index("c")
    base = wid * PER_W

    pltpu.sync_copy(tok_hbm.at[pl.ds(pl.multiple_of(base, 8), PER_W)], idx_v)

    def fire(rb, sem, c):
        off = c * C
        for j in range(C // 128):
            pltpu.async_copy(table_hbm.at[idx_v.at[pl.ds(off + j * 128, 128)]],
                             rb.at[pl.ds(j * 128, 128)], sem)

    def drain(rb, sem):
        pltpu.make_async_copy(table_hbm.at[pl.ds(0, C)], rb, sem).wait()

    def writeback(rb, c):
        row = pl.multiple_of(base + c * C, 8)
        pltpu.sync_copy(rb.at[:, pl.ds(0, EMB)], out_hbm.at[pl.ds(row, C)])

    fire(r0, g0, 0)

    def body(i, carry):
        c0 = 2 * i
        c1 = c0 + 1
        fire(r1, g1, c1)
        drain(r0, g0)
        writeback(r0, c0)

        @pl.when(i < NPAIR - 1)
        def _():
            fire(r0, g0, c0 + 2)

        drain(r1, g1)
        writeback(r1, c1)
        return carry

    lax.fori_loop(0, NPAIR, body, 0)


@jax.jit
def _embed(tokens, table):
    tT = table.T
    wide = pl.pallas_call(
        _transpose_body,
        grid=((NT + CB - 1) // CB,),
        in_specs=[pl.BlockSpec((EMB, CB), lambda b: (0, b))],
        out_specs=pl.BlockSpec((CB, 128), lambda b: (b, 0)),
        out_shape=jax.ShapeDtypeStruct((NT, 128), jnp.float32),
    )(tT)
    tok1d = tokens.reshape(B).astype(jnp.int32)
    mesh = plsc.VectorSubcoreMesh(core_axis_name="c", subcore_axis_name="s")
    run = functools.partial(
        pl.kernel,
        out_type=jax.ShapeDtypeStruct((B, EMB), jnp.float32),
        mesh=mesh,
        scratch_types=[
            pltpu.VMEM((PER_W,), jnp.int32),
            pltpu.VMEM((C, 128), jnp.float32),
            pltpu.VMEM((C, 128), jnp.float32),
            pltpu.SemaphoreType.DMA,
            pltpu.SemaphoreType.DMA,
        ],
        compiler_params=pltpu.CompilerParams(use_tc_tiling_on_sc=False),
    )(_body)
    return run(tok1d, wide).reshape(N0, N1, EMB)


def kernel(tokens, table):
    return _embed(tokens, table)


# speedup vs baseline: 1.1237x; 1.1237x over previous
import functools
import math

import jax
import jax.numpy as jnp
from jax import lax
from jax.experimental import pallas as pl
from jax.experimental.pallas import tpu as pltpu
from jax.experimental.pallas import tpu_sc as plsc

EMB = 64
SCALE = math.sqrt(EMB)

N0 = 16384
N1 = 50
B = N0 * N1
NT = 1000000
NC, NS, L = 2, 16, 16
NW = NC * NS
PER_W = B // NW
C = 256
NPAIR = PER_W // C // 2

CB = 2048


def _transpose_body(tT_ref, out_ref):
    blk = tT_ref[...]
    out_ref[:, 0:EMB] = blk.T * jnp.float32(SCALE)


def _body(tok_hbm, table_hbm, out_hbm, idx_v, r0, r1, g0, g1):
    wid = lax.axis_index("s") * NC + lax.axis_index("c")
    base = wid * PER_W

    pltpu.sync_copy(tok_hbm.at[pl.ds(pl.multiple_of(base, 8), PER_W)], idx_v)

    def fire(rb, sem, c):
        off = c * C
        for j in range(C // 128):
            pltpu.async_copy(table_hbm.at[idx_v.at[pl.ds(off + j * 128, 128)]],
                             rb.at[pl.ds(j * 128, 128)], sem)

    def drain(rb, sem):
        pltpu.make_async_copy(table_hbm.at[pl.ds(0, C)], rb, sem).wait()

    def writeback(rb, c):
        row = pl.multiple_of(base + c * C, 8)
        pltpu.sync_copy(rb.at[:, pl.ds(0, EMB)], out_hbm.at[pl.ds(row, C)])

    fire(r0, g0, 0)

    def body(i, carry):
        c0 = 2 * i
        c1 = c0 + 1
        fire(r1, g1, c1)
        drain(r0, g0)
        writeback(r0, c0)

        @pl.when(i < NPAIR - 1)
        def _():
            fire(r0, g0, c0 + 2)

        drain(r1, g1)
        writeback(r1, c1)
        return carry

    lax.fori_loop(0, NPAIR, body, 0)


@jax.jit
def _embed(tokens, table):
    tT = table.T
    wide = pl.pallas_call(
        _transpose_body,
        grid=((NT + CB - 1) // CB,),
        in_specs=[pl.BlockSpec((EMB, CB), lambda b: (0, b))],
        out_specs=pl.BlockSpec((CB, 128), lambda b: (b, 0)),
        out_shape=jax.ShapeDtypeStruct((NT, 128), jnp.float32),
    )(tT)
    tok1d = tokens.reshape(B).astype(jnp.int32)
    mesh = plsc.VectorSubcoreMesh(core_axis_name="c", subcore_axis_name="s")
    run = functools.partial(
        pl.kernel,
        out_type=jax.ShapeDtypeStruct((B, EMB), jnp.float32),
        mesh=mesh,
        scratch_types=[
            pltpu.VMEM((PER_W,), jnp.int32),
            pltpu.VMEM((C, 128), jnp.float32),
            pltpu.VMEM((C, 128), jnp.float32),
            pltpu.SemaphoreType.DMA,
            pltpu.SemaphoreType.DMA,
        ],
        compiler_params=pltpu.CompilerParams(use_tc_tiling_on_sc=False),
    )(_body)
    return run(tok1d, wide).reshape(N0, N1, EMB)


def kernel(tokens, table):
    return _embed(tokens, table)


# speedup vs baseline: 1.3714x; 1.2204x over previous
import functools
import math

import jax
import jax.numpy as jnp
from jax import lax
from jax.experimental import pallas as pl
from jax.experimental.pallas import tpu as pltpu
from jax.experimental.pallas import tpu_sc as plsc

EMB = 64
SCALE = math.sqrt(EMB)

N0 = 16384
N1 = 50
B = N0 * N1
NT = 1000000
NC, NS, L = 2, 16, 16
NW = NC * NS
PER_W = B // NW
C = 256
NPAIR = PER_W // C // 2

CB = 16384


def _transpose_body(tT_ref, out_ref):
    blk = tT_ref[...]
    out_ref[:, 0:EMB] = blk.T * jnp.float32(SCALE)


def _body(tok_hbm, table_hbm, out_hbm, idx_v, r0, r1, g0, g1):
    wid = lax.axis_index("s") * NC + lax.axis_index("c")
    base = wid * PER_W

    pltpu.sync_copy(tok_hbm.at[pl.ds(pl.multiple_of(base, 8), PER_W)], idx_v)

    def fire(rb, sem, c):
        off = c * C
        for j in range(C // 128):
            pltpu.async_copy(table_hbm.at[idx_v.at[pl.ds(off + j * 128, 128)]],
                             rb.at[pl.ds(j * 128, 128)], sem)

    def drain(rb, sem):
        pltpu.make_async_copy(table_hbm.at[pl.ds(0, C)], rb, sem).wait()

    def writeback(rb, c):
        row = pl.multiple_of(base + c * C, 8)
        pltpu.sync_copy(rb.at[:, pl.ds(0, EMB)], out_hbm.at[pl.ds(row, C)])

    fire(r0, g0, 0)

    def body(i, carry):
        c0 = 2 * i
        c1 = c0 + 1
        fire(r1, g1, c1)
        drain(r0, g0)
        writeback(r0, c0)

        @pl.when(i < NPAIR - 1)
        def _():
            fire(r0, g0, c0 + 2)

        drain(r1, g1)
        writeback(r1, c1)
        return carry

    lax.fori_loop(0, NPAIR, body, 0)


@jax.jit
def _embed(tokens, table):
    tT = table.T
    wide = pl.pallas_call(
        _transpose_body,
        grid=((NT + CB - 1) // CB,),
        in_specs=[pl.BlockSpec((EMB, CB), lambda b: (0, b))],
        out_specs=pl.BlockSpec((CB, 128), lambda b: (b, 0)),
        out_shape=jax.ShapeDtypeStruct((NT, 128), jnp.float32),
    )(tT)
    tok1d = tokens.reshape(B).astype(jnp.int32)
    mesh = plsc.VectorSubcoreMesh(core_axis_name="c", subcore_axis_name="s")
    run = functools.partial(
        pl.kernel,
        out_type=jax.ShapeDtypeStruct((B, EMB), jnp.float32),
        mesh=mesh,
        scratch_types=[
            pltpu.VMEM((PER_W,), jnp.int32),
            pltpu.VMEM((C, 128), jnp.float32),
            pltpu.VMEM((C, 128), jnp.float32),
            pltpu.SemaphoreType.DMA,
            pltpu.SemaphoreType.DMA,
        ],
        compiler_params=pltpu.CompilerParams(use_tc_tiling_on_sc=False),
    )(_body)
    return run(tok1d, wide).reshape(N0, N1, EMB)


def kernel(tokens, table):
    return _embed(tokens, table)


# speedup vs baseline: 1.4016x; 1.0220x over previous
import functools
import math

import jax
import jax.numpy as jnp
from jax import lax
from jax.experimental import pallas as pl
from jax.experimental.pallas import tpu as pltpu
from jax.experimental.pallas import tpu_sc as plsc

EMB = 64
SCALE = math.sqrt(EMB)

N0 = 16384
N1 = 50
B = N0 * N1
NT = 1000000
NC, NS, L = 2, 16, 16
NW = NC * NS
PER_W = B // NW
C = 256
NPAIR = PER_W // C // 2

CB = 16384


def _transpose_body(tT_ref, out_ref):
    blk = tT_ref[...]
    out_ref[:, 0:EMB] = blk.T * jnp.float32(SCALE)


N0B = 4096
NBLK = N0 // N0B


def _untranspose_body(in_ref, out_ref):
    x = in_ref[0]
    out_ref[0, :, 0:N0B // 2] = x[:, 0:EMB].T
    out_ref[0, :, N0B // 2:N0B] = x[:, EMB:128].T


def _body(tok_hbm, table_hbm, out_hbm, idx_v, r0, r1, g0, g1):
    wid = lax.axis_index("s") * NC + lax.axis_index("c")
    base = wid * PER_W

    pltpu.sync_copy(tok_hbm.at[pl.ds(pl.multiple_of(base, 8), PER_W)], idx_v)

    def fire(rb, sem, c):
        off = c * C
        for j in range(C // 128):
            pltpu.async_copy(table_hbm.at[idx_v.at[pl.ds(off + j * 128, 128)]],
                             rb.at[pl.ds(j * 128, 128)], sem)

    def drain(rb, sem):
        pltpu.make_async_copy(table_hbm.at[pl.ds(0, C)], rb, sem).wait()

    def writeback(rb, c):
        row = pl.multiple_of(base + c * C, 8)
        pltpu.sync_copy(rb.at[:, pl.ds(0, EMB)], out_hbm.at[pl.ds(row, C)])

    fire(r0, g0, 0)

    def body(i, carry):
        c0 = 2 * i
        c1 = c0 + 1
        fire(r1, g1, c1)
        drain(r0, g0)
        writeback(r0, c0)

        @pl.when(i < NPAIR - 1)
        def _():
            fire(r0, g0, c0 + 2)

        drain(r1, g1)
        writeback(r1, c1)
        return carry

    lax.fori_loop(0, NPAIR, body, 0)


@jax.jit
def _embed(tokens, table):
    tT = table.T
    wide = pl.pallas_call(
        _transpose_body,
        grid=((NT + CB - 1) // CB,),
        in_specs=[pl.BlockSpec((EMB, CB), lambda b: (0, b))],
        out_specs=pl.BlockSpec((CB, 128), lambda b: (b, 0)),
        out_shape=jax.ShapeDtypeStruct((NT, 128), jnp.float32),
    )(tT)
    tok1d = (tokens.T.astype(jnp.int32)
             .reshape(N1, NBLK, 2, N0B // 2)
             .transpose(0, 1, 3, 2)
             .reshape(B))
    mesh = plsc.VectorSubcoreMesh(core_axis_name="c", subcore_axis_name="s")
    run = functools.partial(
        pl.kernel,
        out_type=jax.ShapeDtypeStruct((B, EMB), jnp.float32),
        mesh=mesh,
        scratch_types=[
            pltpu.VMEM((PER_W,), jnp.int32),
            pltpu.VMEM((C, 128), jnp.float32),
            pltpu.VMEM((C, 128), jnp.float32),
            pltpu.SemaphoreType.DMA,
            pltpu.SemaphoreType.DMA,
        ],
        compiler_params=pltpu.CompilerParams(use_tc_tiling_on_sc=False),
    )(_body)
    out2 = run(tok1d, wide)
    o3 = pl.pallas_call(
        _untranspose_body,
        grid=(N1, N0 // N0B),
        in_specs=[pl.BlockSpec((1, N0B // 2, 128), lambda j, k: (j, k, 0))],
        out_specs=pl.BlockSpec((1, EMB, N0B), lambda j, k: (j, 0, k)),
        out_shape=jax.ShapeDtypeStruct((N1, EMB, N0), jnp.float32),
    )(out2.reshape(N1, N0 // 2, 128))
    return jnp.transpose(o3, (2, 0, 1))


def kernel(tokens, table):
    return _embed(tokens, table)


# speedup vs baseline: 1.4836x; 1.0585x over previous
import functools
import math

import jax
import jax.numpy as jnp
from jax import lax
from jax.experimental import pallas as pl
from jax.experimental.pallas import tpu as pltpu
from jax.experimental.pallas import tpu_sc as plsc

EMB = 64
SCALE = math.sqrt(EMB)

N0 = 16384
N1 = 50
B = N0 * N1
NT = 1000000
NC, NS, L = 2, 16, 16
NW = NC * NS
PER_W = B // NW
C = 256
NPAIR = PER_W // C // 2

CB = 16384


def _transpose_body(tT_ref, out_ref):
    blk = tT_ref[...]
    out_ref[:, 0:EMB] = blk.T * jnp.float32(SCALE)


N0B = 8192
NBLK = N0 // N0B


def _untranspose_body(in_ref, out_ref):
    x = in_ref[0]
    out_ref[0, :, 0:N0B // 2] = x[:, 0:EMB].T
    out_ref[0, :, N0B // 2:N0B] = x[:, EMB:128].T


def _body(tok_hbm, table_hbm, out_hbm, idx_v, r0, r1, g0, g1):
    wid = lax.axis_index("s") * NC + lax.axis_index("c")
    base = wid * PER_W

    pltpu.sync_copy(tok_hbm.at[pl.ds(pl.multiple_of(base, 8), PER_W)], idx_v)

    def fire(rb, sem, c):
        off = c * C
        for j in range(C // 128):
            pltpu.async_copy(table_hbm.at[idx_v.at[pl.ds(off + j * 128, 128)]],
                             rb.at[pl.ds(j * 128, 128)], sem)

    def drain(rb, sem):
        pltpu.make_async_copy(table_hbm.at[pl.ds(0, C)], rb, sem).wait()

    def writeback(rb, c):
        row = pl.multiple_of(base + c * C, 8)
        pltpu.sync_copy(rb.at[:, pl.ds(0, EMB)], out_hbm.at[pl.ds(row, C)])

    fire(r0, g0, 0)

    def body(i, carry):
        c0 = 2 * i
        c1 = c0 + 1
        fire(r1, g1, c1)
        drain(r0, g0)
        writeback(r0, c0)

        @pl.when(i < NPAIR - 1)
        def _():
            fire(r0, g0, c0 + 2)

        drain(r1, g1)
        writeback(r1, c1)
        return carry

    lax.fori_loop(0, NPAIR, body, 0)


@jax.jit
def _embed(tokens, table):
    tT = table.T
    wide = pl.pallas_call(
        _transpose_body,
        grid=((NT + CB - 1) // CB,),
        in_specs=[pl.BlockSpec((EMB, CB), lambda b: (0, b))],
        out_specs=pl.BlockSpec((CB, 128), lambda b: (b, 0)),
        out_shape=jax.ShapeDtypeStruct((NT, 128), jnp.float32),
    )(tT)
    tok1d = (tokens.T.astype(jnp.int32)
             .reshape(N1, NBLK, 2, N0B // 2)
             .transpose(0, 1, 3, 2)
             .reshape(B))
    mesh = plsc.VectorSubcoreMesh(core_axis_name="c", subcore_axis_name="s")
    run = functools.partial(
        pl.kernel,
        out_type=jax.ShapeDtypeStruct((B, EMB), jnp.float32),
        mesh=mesh,
        scratch_types=[
            pltpu.VMEM((PER_W,), jnp.int32),
            pltpu.VMEM((C, 128), jnp.float32),
            pltpu.VMEM((C, 128), jnp.float32),
            pltpu.SemaphoreType.DMA,
            pltpu.SemaphoreType.DMA,
        ],
        compiler_params=pltpu.CompilerParams(use_tc_tiling_on_sc=False),
    )(_body)
    out2 = run(tok1d, wide)
    o3 = pl.pallas_call(
        _untranspose_body,
        grid=(N1, N0 // N0B),
        in_specs=[pl.BlockSpec((1, N0B // 2, 128), lambda j, k: (j, k, 0))],
        out_specs=pl.BlockSpec((1, EMB, N0B), lambda j, k: (j, 0, k)),
        out_shape=jax.ShapeDtypeStruct((N1, EMB, N0), jnp.float32),
    )(out2.reshape(N1, N0 // 2, 128))
    return jnp.transpose(o3, (2, 0, 1))


def kernel(tokens, table):
    return _embed(tokens, table)


# speedup vs baseline: 1.5292x; 1.0307x over previous
import functools
import math

import jax
import jax.numpy as jnp
from jax import lax
from jax.experimental import pallas as pl
from jax.experimental.pallas import tpu as pltpu
from jax.experimental.pallas import tpu_sc as plsc

EMB = 64
SCALE = math.sqrt(EMB)

N0 = 16384
N1 = 50
B = N0 * N1
NT = 1000000
NC, NS, L = 2, 16, 16
NW = NC * NS
PER_W = B // NW
C = 256
NPAIR = PER_W // C // 2

CB = 16384


def _transpose_body(tT_ref, out_ref):
    blk = tT_ref[...]
    out_ref[:, 0:EMB] = blk.T * jnp.float32(SCALE)


N0B = 16384
NBLK = N0 // N0B


def _untranspose_body(in_ref, out_ref):
    x = in_ref[0]
    out_ref[0, :, 0:N0B // 2] = x[:, 0:EMB].T
    out_ref[0, :, N0B // 2:N0B] = x[:, EMB:128].T


def _body(tok_hbm, table_hbm, out_hbm, idx_v, r0, r1, g0, g1):
    wid = lax.axis_index("s") * NC + lax.axis_index("c")
    base = wid * PER_W

    pltpu.sync_copy(tok_hbm.at[pl.ds(pl.multiple_of(base, 8), PER_W)], idx_v)

    def fire(rb, sem, c):
        off = c * C
        for j in range(C // 128):
            pltpu.async_copy(table_hbm.at[idx_v.at[pl.ds(off + j * 128, 128)]],
                             rb.at[pl.ds(j * 128, 128)], sem)

    def drain(rb, sem):
        pltpu.make_async_copy(table_hbm.at[pl.ds(0, C)], rb, sem).wait()

    def writeback(rb, c):
        row = pl.multiple_of(base + c * C, 8)
        pltpu.sync_copy(rb.at[:, pl.ds(0, EMB)], out_hbm.at[pl.ds(row, C)])

    fire(r0, g0, 0)

    def body(i, carry):
        c0 = 2 * i
        c1 = c0 + 1
        fire(r1, g1, c1)
        drain(r0, g0)
        writeback(r0, c0)

        @pl.when(i < NPAIR - 1)
        def _():
            fire(r0, g0, c0 + 2)

        drain(r1, g1)
        writeback(r1, c1)
        return carry

    lax.fori_loop(0, NPAIR, body, 0)


@jax.jit
def _embed(tokens, table):
    tT = table.T
    wide = pl.pallas_call(
        _transpose_body,
        grid=((NT + CB - 1) // CB,),
        in_specs=[pl.BlockSpec((EMB, CB), lambda b: (0, b))],
        out_specs=pl.BlockSpec((CB, 128), lambda b: (b, 0)),
        out_shape=jax.ShapeDtypeStruct((NT, 128), jnp.float32),
    )(tT)
    tok1d = (tokens.T.astype(jnp.int32)
             .reshape(N1, NBLK, 2, N0B // 2)
             .transpose(0, 1, 3, 2)
             .reshape(B))
    mesh = plsc.VectorSubcoreMesh(core_axis_name="c", subcore_axis_name="s")
    run = functools.partial(
        pl.kernel,
        out_type=jax.ShapeDtypeStruct((B, EMB), jnp.float32),
        mesh=mesh,
        scratch_types=[
            pltpu.VMEM((PER_W,), jnp.int32),
            pltpu.VMEM((C, 128), jnp.float32),
            pltpu.VMEM((C, 128), jnp.float32),
            pltpu.SemaphoreType.DMA,
            pltpu.SemaphoreType.DMA,
        ],
        compiler_params=pltpu.CompilerParams(use_tc_tiling_on_sc=False),
    )(_body)
    out2 = run(tok1d, wide)
    o3 = pl.pallas_call(
        _untranspose_body,
        grid=(N1, N0 // N0B),
        in_specs=[pl.BlockSpec((1, N0B // 2, 128), lambda j, k: (j, k, 0))],
        out_specs=pl.BlockSpec((1, EMB, N0B), lambda j, k: (j, 0, k)),
        out_shape=jax.ShapeDtypeStruct((N1, EMB, N0), jnp.float32),
    )(out2.reshape(N1, N0 // 2, 128))
    return jnp.transpose(o3, (2, 0, 1))


def kernel(tokens, table):
    return _embed(tokens, table)
